# baseline (speedup 1.0000x reference)
import functools

import jax
import jax.numpy as jnp
from jax import lax
from jax.experimental import pallas as pl
from jax.experimental.pallas import tpu as pltpu
from jax.experimental.pallas import tpu_sc as plsc

B = 4096
SEQ = 200
D = 32
VOCAB = 1000000
LANES = 16
NC = 2
NS = 16
NW = NC * NS
RPW = B // NW
CH1 = 104
CH2 = 96

NTC = 7813
LAST_TC = NTC - 1
LAST_ROWS = VOCAB - LAST_TC * 128


def _transpose_block(in_v, tr_v, nsub):
    lane = lax.iota(jnp.int32, LANES)
    for sub in range(nsub):

        @plsc.parallel_loop(0, D * 8, unroll=8)
        def _(k):
            d = k // 8
            g = k % 8
            val = plsc.load_gather(
                in_v, [jnp.broadcast_to(d, (LANES,)),
                       lane + (sub * 128 + g * 16)])
            plsc.store_scatter(
                tr_v, [lane * D + (sub * 4096 + g * 16 * D) + d], val)


GRP = 4
NGRP = (NTC - 1) // GRP
GW = GRP * 128
GWORDS = GRP * 4096


def _phase_a_body(wtT_hbm, out_hbm, in0, in1, tr0, tr1,
                  isem0, isem1, osem0, osem1):
    wid = lax.axis_index("s") * NC + lax.axis_index("c")
    n = (NGRP - wid + NW - 1) // NW

    def start_in(i, in_v, isem):
        g = wid + NW * i
        off = pl.multiple_of(g * GW, GW)
        pltpu.async_copy(wtT_hbm.at[:, pl.ds(off, GW)], in_v, isem)

    def wait_in(i, in_v, isem):
        g = wid + NW * i
        off = pl.multiple_of(g * GW, GW)
        pltpu.make_async_copy(wtT_hbm.at[:, pl.ds(off, GW)], in_v,
                              isem).wait()

    def start_out(i, tr_v, osem):
        g = wid + NW * i
        off = pl.multiple_of(g * GWORDS, GWORDS)
        pltpu.async_copy(tr_v, out_hbm.at[pl.ds(off, GWORDS)], osem)

    def wait_out(i, tr_v, osem):
        g = wid + NW * i
        off = pl.multiple_of(g * GWORDS, GWORDS)
        pltpu.make_async_copy(tr_v, out_hbm.at[pl.ds(off, GWORDS)],
                              osem).wait()

    @pl.when(0 < n)
    def _():
        start_in(0, in0, isem0)

    def pair(i2, _):
        i0 = 2 * i2
        i1 = i0 + 1

        @pl.when(i1 < n)
        def _():
            start_in(i1, in1, isem1)

        @pl.when(i0 < n)
        def _():
            wait_in(i0, in0, isem0)

            @pl.when(i0 >= 2)
            def _():
                wait_out(i0 - 2, tr0, osem0)

            _transpose_block(in0, tr0, GRP)
            start_out(i0, tr0, osem0)

        @pl.when(i0 + 2 < n)
        def _():
            start_in(i0 + 2, in0, isem0)

        @pl.when(i1 < n)
        def _():
            wait_in(i1, in1, isem1)

            @pl.when(i1 >= 2)
            def _():
                wait_out(i1 - 2, tr1, osem1)

            _transpose_block(in1, tr1, GRP)
            start_out(i1, tr1, osem1)

        return 0

    lax.fori_loop(0, (NGRP // NW + 2) // 2, pair, 0)

    for back in (2, 1):
        idx = n - back

        @pl.when(idx % 2 == 0)
        def _():
            wait_out(idx, tr0, osem0)

        @pl.when(idx % 2 == 1)
        def _():
            wait_out(idx, tr1, osem1)

    @pl.when(wid == 0)
    def _():
        off = pl.multiple_of(wid + LAST_TC * 128, 128)
        pltpu.sync_copy(wtT_hbm.at[:, pl.ds(off, 128)],
                        in0.at[:, pl.ds(0, 128)])
        _transpose_block(in0, tr0, 1)
        pltpu.sync_copy(tr0.at[pl.ds(0, LAST_ROWS * D)],
                        out_hbm.at[pl.ds(LAST_TC * 4096, LAST_ROWS * D)])


@functools.partial(
    pl.kernel,
    out_type=jax.ShapeDtypeStruct((VOCAB * D,), jnp.float32),
    mesh=plsc.VectorSubcoreMesh(core_axis_name="c", subcore_axis_name="s"),
    compiler_params=pltpu.CompilerParams(
        use_tc_tiling_on_sc=True, needs_layout_passes=False),
    scratch_types=[
        pltpu.VMEM((D, GW), jnp.float32),
        pltpu.VMEM((D, GW), jnp.float32),
        pltpu.VMEM((GWORDS,), jnp.float32),
        pltpu.VMEM((GWORDS,), jnp.float32),
        pltpu.SemaphoreType.DMA,
        pltpu.SemaphoreType.DMA,
        pltpu.SemaphoreType.DMA,
        pltpu.SemaphoreType.DMA,
    ],
)
def _relayout(wtT_hbm, out_hbm, in0, in1, tr0, tr1, s0, s1, s2, s3):
    _phase_a_body(wtT_hbm, out_hbm, in0, in1, tr0, tr1, s0, s1, s2, s3)


def _phase_b_body(text_hbm, lens_hbm, table_hbm, out_hbm,
                  text_v, lens_v, rows0, rows1, out_v, sem0, sem1):
    wid = lax.axis_index("s") * NC + lax.axis_index("c")
    base = wid * RPW

    pltpu.sync_copy(text_hbm.at[pl.ds(base, RPW), :], text_v)
    pltpu.sync_copy(lens_hbm.at[pl.ds(base, RPW)], lens_v)

    def fire(r, rows_v, sem):
        idx1 = text_v.at[r, pl.ds(0, CH1)]
        idx2 = text_v.at[r, pl.ds(CH1, CH2)]
        pltpu.async_copy(table_hbm.at[idx1], rows_v.at[pl.ds(0, CH1), :], sem)
        pltpu.async_copy(table_hbm.at[idx2], rows_v.at[pl.ds(CH1, CH2), :], sem)

    def wait(r, rows_v, sem):
        idx1 = text_v.at[r, pl.ds(0, CH1)]
        idx2 = text_v.at[r, pl.ds(CH1, CH2)]
        pltpu.make_async_copy(table_hbm.at[idx1],
                              rows_v.at[pl.ds(0, CH1), :], sem).wait()
        pltpu.make_async_copy(table_hbm.at[idx2],
                              rows_v.at[pl.ds(CH1, CH2), :], sem).wait()

    def accumulate(r, rows_v):
        len_vec = plsc.load_gather(lens_v, [jnp.broadcast_to(r, (LANES,))])
        len_s = jnp.max(len_vec)
        n8 = len_s // 8

        def chunk_body(c, carry):
            a0, a1 = carry
            t0 = c * 8
            for u in range(8):
                a0 = a0 + rows_v[t0 + u, 0:16]
                a1 = a1 + rows_v[t0 + u, 16:32]
            return a0, a1

        zero = jnp.zeros((LANES,), jnp.float32)
        acc0, acc1 = lax.fori_loop(0, n8, chunk_body, (zero, zero))

        def rem_body(t, carry):
            a0, a1 = carry
            return a0 + rows_v[t, 0:16], a1 + rows_v[t, 16:32]

        acc0, acc1 = lax.fori_loop(n8 * 8, len_s, rem_body, (acc0, acc1))

        inv = 1.0 / len_vec.astype(jnp.float32)
        out_v[r, 0:16] = acc0 * inv
        out_v[r, 16:32] = acc1 * inv

    fire(0, rows0, sem0)

    def outer(i, _):
        r0 = 2 * i
        r1 = 2 * i + 1
        fire(r1, rows1, sem1)
        wait(r0, rows0, sem0)
        accumulate(r0, rows0)

        @pl.when(i < RPW // 2 - 1)
        def _():
            fire(r0 + 2, rows0, sem0)

        wait(r1, rows1, sem1)
        accumulate(r1, rows1)
        return 0

    lax.fori_loop(0, RPW // 2, outer, 0)

    pltpu.sync_copy(out_v, out_hbm.at[pl.ds(base, RPW), :])


@functools.partial(
    pl.kernel,
    out_type=jax.ShapeDtypeStruct((B, D), jnp.float32),
    mesh=plsc.VectorSubcoreMesh(core_axis_name="c", subcore_axis_name="s"),
    compiler_params=pltpu.CompilerParams(
        use_tc_tiling_on_sc=False, needs_layout_passes=False),
    scratch_types=[
        pltpu.VMEM((RPW, SEQ), jnp.int32),
        pltpu.VMEM((RPW,), jnp.int32),
        pltpu.VMEM((SEQ, D), jnp.float32),
        pltpu.VMEM((SEQ, D), jnp.float32),
        pltpu.VMEM((RPW, D), jnp.float32),
        pltpu.SemaphoreType.DMA,
        pltpu.SemaphoreType.DMA,
    ],
)
def _encode(text_hbm, lens_hbm, table_hbm, out_hbm,
            text_v, lens_v, rows0, rows1, out_v, sem0, sem1):
    _phase_b_body(text_hbm, lens_hbm, table_hbm, out_hbm,
                  text_v, lens_v, rows0, rows1, out_v, sem0, sem1)


def kernel(text, text_len, emb_weight):
    wt_lin = _relayout(emb_weight.T).reshape(VOCAB, D)
    return _encode(text.astype(jnp.int32), text_len, wt_lin)


# speedup vs baseline: 3.5348x; 3.5348x over previous
import functools

import jax
import jax.numpy as jnp
from jax import lax
from jax.experimental import pallas as pl
from jax.experimental.pallas import tpu as pltpu
from jax.experimental.pallas import tpu_sc as plsc

B = 4096
SEQ = 200
D = 32
VOCAB = 1000000
LANES = 16
NC = 2
NS = 16
NW = NC * NS
RPW = B // NW
CH1 = 104
CH2 = 96

NTC = 7813
LAST_TC = NTC - 1
LAST_ROWS = VOCAB - LAST_TC * 128


def _transpose_block(in_v, tr_v, nsub):
    lane = lax.iota(jnp.int32, LANES)
    for sub in range(nsub):

        @plsc.parallel_loop(0, D * 8, unroll=8)
        def _(k):
            d0 = k // 8
            g = k % 8
            dvec = (d0 + lane) & (D - 1)
            val = plsc.load_gather(
                in_v, [dvec, lane + (sub * 128 + g * 16)])
            plsc.store_scatter(
                tr_v, [lane * D + (sub * 4096 + g * 16 * D) + dvec], val)


GRP = 4
NGRP = (NTC - 1) // GRP
GW = GRP * 128
GWORDS = GRP * 4096


def _phase_a_body(wtT_hbm, out_hbm, in0, in1, tr0, tr1,
                  isem0, isem1, osem0, osem1):
    wid = lax.axis_index("s") * NC + lax.axis_index("c")
    n = (NGRP - wid + NW - 1) // NW

    def start_in(i, in_v, isem):
        g = wid + NW * i
        off = pl.multiple_of(g * GW, GW)
        pltpu.async_copy(wtT_hbm.at[:, pl.ds(off, GW)], in_v, isem)

    def wait_in(i, in_v, isem):
        g = wid + NW * i
        off = pl.multiple_of(g * GW, GW)
        pltpu.make_async_copy(wtT_hbm.at[:, pl.ds(off, GW)], in_v,
                              isem).wait()

    def start_out(i, tr_v, osem):
        g = wid + NW * i
        off = pl.multiple_of(g * GWORDS, GWORDS)
        pltpu.async_copy(tr_v, out_hbm.at[pl.ds(off, GWORDS)], osem)

    def wait_out(i, tr_v, osem):
        g = wid + NW * i
        off = pl.multiple_of(g * GWORDS, GWORDS)
        pltpu.make_async_copy(tr_v, out_hbm.at[pl.ds(off, GWORDS)],
                              osem).wait()

    @pl.when(0 < n)
    def _():
        start_in(0, in0, isem0)

    def pair(i2, _):
        i0 = 2 * i2
        i1 = i0 + 1

        @pl.when(i1 < n)
        def _():
            start_in(i1, in1, isem1)

        @pl.when(i0 < n)
        def _():
            wait_in(i0, in0, isem0)

            @pl.when(i0 >= 2)
            def _():
                wait_out(i0 - 2, tr0, osem0)

            _transpose_block(in0, tr0, GRP)
            start_out(i0, tr0, osem0)

        @pl.when(i0 + 2 < n)
        def _():
            start_in(i0 + 2, in0, isem0)

        @pl.when(i1 < n)
        def _():
            wait_in(i1, in1, isem1)

            @pl.when(i1 >= 2)
            def _():
                wait_out(i1 - 2, tr1, osem1)

            _transpose_block(in1, tr1, GRP)
            start_out(i1, tr1, osem1)

        return 0

    lax.fori_loop(0, (NGRP // NW + 2) // 2, pair, 0)

    for back in (2, 1):
        idx = n - back

        @pl.when(idx % 2 == 0)
        def _():
            wait_out(idx, tr0, osem0)

        @pl.when(idx % 2 == 1)
        def _():
            wait_out(idx, tr1, osem1)

    @pl.when(wid == 0)
    def _():
        off = pl.multiple_of(wid + LAST_TC * 128, 128)
        pltpu.sync_copy(wtT_hbm.at[:, pl.ds(off, 128)],
                        in0.at[:, pl.ds(0, 128)])
        _transpose_block(in0, tr0, 1)
        pltpu.sync_copy(tr0.at[pl.ds(0, LAST_ROWS * D)],
                        out_hbm.at[pl.ds(LAST_TC * 4096, LAST_ROWS * D)])


@functools.partial(
    pl.kernel,
    out_type=jax.ShapeDtypeStruct((VOCAB * D,), jnp.float32),
    mesh=plsc.VectorSubcoreMesh(core_axis_name="c", subcore_axis_name="s"),
    compiler_params=pltpu.CompilerParams(
        use_tc_tiling_on_sc=True, needs_layout_passes=False),
    scratch_types=[
        pltpu.VMEM((D, GW), jnp.float32),
        pltpu.VMEM((D, GW), jnp.float32),
        pltpu.VMEM((GWORDS,), jnp.float32),
        pltpu.VMEM((GWORDS,), jnp.float32),
        pltpu.SemaphoreType.DMA,
        pltpu.SemaphoreType.DMA,
        pltpu.SemaphoreType.DMA,
        pltpu.SemaphoreType.DMA,
    ],
)
def _relayout(wtT_hbm, out_hbm, in0, in1, tr0, tr1, s0, s1, s2, s3):
    _phase_a_body(wtT_hbm, out_hbm, in0, in1, tr0, tr1, s0, s1, s2, s3)


def _phase_b_body(text_hbm, lens_hbm, table_hbm, out_hbm,
                  text_v, lens_v, rows0, rows1, out_v, sem0, sem1):
    wid = lax.axis_index("s") * NC + lax.axis_index("c")
    base = wid * RPW

    pltpu.sync_copy(text_hbm.at[pl.ds(base, RPW), :], text_v)
    pltpu.sync_copy(lens_hbm.at[pl.ds(base, RPW)], lens_v)

    def fire(r, rows_v, sem):
        idx1 = text_v.at[r, pl.ds(0, CH1)]
        idx2 = text_v.at[r, pl.ds(CH1, CH2)]
        pltpu.async_copy(table_hbm.at[idx1], rows_v.at[pl.ds(0, CH1), :], sem)
        pltpu.async_copy(table_hbm.at[idx2], rows_v.at[pl.ds(CH1, CH2), :], sem)

    def wait(r, rows_v, sem):
        idx1 = text_v.at[r, pl.ds(0, CH1)]
        idx2 = text_v.at[r, pl.ds(CH1, CH2)]
        pltpu.make_async_copy(table_hbm.at[idx1],
                              rows_v.at[pl.ds(0, CH1), :], sem).wait()
        pltpu.make_async_copy(table_hbm.at[idx2],
                              rows_v.at[pl.ds(CH1, CH2), :], sem).wait()

    def accumulate(r, rows_v):
        len_vec = plsc.load_gather(lens_v, [jnp.broadcast_to(r, (LANES,))])
        len_s = jnp.max(len_vec)
        n8 = len_s // 8

        def chunk_body(c, carry):
            a0, a1 = carry
            t0 = c * 8
            for u in range(8):
                a0 = a0 + rows_v[t0 + u, 0:16]
                a1 = a1 + rows_v[t0 + u, 16:32]
            return a0, a1

        zero = jnp.zeros((LANES,), jnp.float32)
        acc0, acc1 = lax.fori_loop(0, n8, chunk_body, (zero, zero))

        def rem_body(t, carry):
            a0, a1 = carry
            return a0 + rows_v[t, 0:16], a1 + rows_v[t, 16:32]

        acc0, acc1 = lax.fori_loop(n8 * 8, len_s, rem_body, (acc0, acc1))

        inv = 1.0 / len_vec.astype(jnp.float32)
        out_v[r, 0:16] = acc0 * inv
        out_v[r, 16:32] = acc1 * inv

    fire(0, rows0, sem0)

    def outer(i, _):
        r0 = 2 * i
        r1 = 2 * i + 1
        fire(r1, rows1, sem1)
        wait(r0, rows0, sem0)
        accumulate(r0, rows0)

        @pl.when(i < RPW // 2 - 1)
        def _():
            fire(r0 + 2, rows0, sem0)

        wait(r1, rows1, sem1)
        accumulate(r1, rows1)
        return 0

    lax.fori_loop(0, RPW // 2, outer, 0)

    pltpu.sync_copy(out_v, out_hbm.at[pl.ds(base, RPW), :])


@functools.partial(
    pl.kernel,
    out_type=jax.ShapeDtypeStruct((B, D), jnp.float32),
    mesh=plsc.VectorSubcoreMesh(core_axis_name="c", subcore_axis_name="s"),
    compiler_params=pltpu.CompilerParams(
        use_tc_tiling_on_sc=False, needs_layout_passes=False),
    scratch_types=[
        pltpu.VMEM((RPW, SEQ), jnp.int32),
        pltpu.VMEM((RPW,), jnp.int32),
        pltpu.VMEM((SEQ, D), jnp.float32),
        pltpu.VMEM((SEQ, D), jnp.float32),
        pltpu.VMEM((RPW, D), jnp.float32),
        pltpu.SemaphoreType.DMA,
        pltpu.SemaphoreType.DMA,
    ],
)
def _encode(text_hbm, lens_hbm, table_hbm, out_hbm,
            text_v, lens_v, rows0, rows1, out_v, sem0, sem1):
    _phase_b_body(text_hbm, lens_hbm, table_hbm, out_hbm,
                  text_v, lens_v, rows0, rows1, out_v, sem0, sem1)


def kernel(text, text_len, emb_weight):
    wt_lin = _relayout(emb_weight.T).reshape(VOCAB, D)
    return _encode(text.astype(jnp.int32), text_len, wt_lin)


# speedup vs baseline: 3.5399x; 1.0015x over previous
import functools

import jax
import jax.numpy as jnp
from jax import lax
from jax.experimental import pallas as pl
from jax.experimental.pallas import tpu as pltpu
from jax.experimental.pallas import tpu_sc as plsc

B = 4096
SEQ = 200
D = 32
VOCAB = 1000000
LANES = 16
NC = 2
NS = 16
NW = NC * NS
RPW = B // NW
CH1 = 104
CH2 = 96

NTC = 7813
LAST_TC = NTC - 1
LAST_ROWS = VOCAB - LAST_TC * 128


def _transpose_block(in_v, tr_v, nsub):
    lane = lax.iota(jnp.int32, LANES)
    for sub in range(nsub):

        @plsc.parallel_loop(0, D * 8, unroll=8)
        def _(k):
            d0 = k // 8
            g = k % 8
            dvec = (d0 + lane) & (D - 1)
            val = plsc.load_gather(
                in_v, [dvec, lane + (sub * 128 + g * 16)])
            plsc.store_scatter(
                tr_v, [lane * D + (sub * 4096 + g * 16 * D) + dvec], val)


GRP = 4
NGRP = (NTC - 1) // GRP
GW = GRP * 128
GWORDS = GRP * 4096


def _phase_a_body(wtT_hbm, out_hbm, in0, in1, tr0, tr1,
                  isem0, isem1, osem0, osem1):
    wid = lax.axis_index("s") * NC + lax.axis_index("c")
    n = (NGRP - wid + NW - 1) // NW

    def start_in(i, in_v, isem):
        g = wid + NW * i
        off = pl.multiple_of(g * GW, GW)
        pltpu.async_copy(wtT_hbm.at[:, pl.ds(off, GW)], in_v, isem)

    def wait_in(i, in_v, isem):
        g = wid + NW * i
        off = pl.multiple_of(g * GW, GW)
        pltpu.make_async_copy(wtT_hbm.at[:, pl.ds(off, GW)], in_v,
                              isem).wait()

    def start_out(i, tr_v, osem):
        g = wid + NW * i
        off = pl.multiple_of(g * GWORDS, GWORDS)
        pltpu.async_copy(tr_v, out_hbm.at[pl.ds(off, GWORDS)], osem)

    def wait_out(i, tr_v, osem):
        g = wid + NW * i
        off = pl.multiple_of(g * GWORDS, GWORDS)
        pltpu.make_async_copy(tr_v, out_hbm.at[pl.ds(off, GWORDS)],
                              osem).wait()

    @pl.when(0 < n)
    def _():
        start_in(0, in0, isem0)

    def pair(i2, _):
        i0 = 2 * i2
        i1 = i0 + 1

        @pl.when(i1 < n)
        def _():
            start_in(i1, in1, isem1)

        @pl.when(i0 < n)
        def _():
            wait_in(i0, in0, isem0)

            @pl.when(i0 >= 2)
            def _():
                wait_out(i0 - 2, tr0, osem0)

            _transpose_block(in0, tr0, GRP)
            start_out(i0, tr0, osem0)

        @pl.when(i0 + 2 < n)
        def _():
            start_in(i0 + 2, in0, isem0)

        @pl.when(i1 < n)
        def _():
            wait_in(i1, in1, isem1)

            @pl.when(i1 >= 2)
            def _():
                wait_out(i1 - 2, tr1, osem1)

            _transpose_block(in1, tr1, GRP)
            start_out(i1, tr1, osem1)

        return 0

    lax.fori_loop(0, (NGRP // NW + 2) // 2, pair, 0)

    for back in (2, 1):
        idx = n - back

        @pl.when(idx % 2 == 0)
        def _():
            wait_out(idx, tr0, osem0)

        @pl.when(idx % 2 == 1)
        def _():
            wait_out(idx, tr1, osem1)

    @pl.when(wid == 0)
    def _():
        off = pl.multiple_of(wid + LAST_TC * 128, 128)
        pltpu.sync_copy(wtT_hbm.at[:, pl.ds(off, 128)],
                        in0.at[:, pl.ds(0, 128)])
        _transpose_block(in0, tr0, 1)
        pltpu.sync_copy(tr0.at[pl.ds(0, LAST_ROWS * D)],
                        out_hbm.at[pl.ds(LAST_TC * 4096, LAST_ROWS * D)])


@functools.partial(
    pl.kernel,
    out_type=jax.ShapeDtypeStruct((VOCAB * D,), jnp.float32),
    mesh=plsc.VectorSubcoreMesh(core_axis_name="c", subcore_axis_name="s"),
    compiler_params=pltpu.CompilerParams(
        use_tc_tiling_on_sc=True, needs_layout_passes=False),
    scratch_types=[
        pltpu.VMEM((D, GW), jnp.float32),
        pltpu.VMEM((D, GW), jnp.float32),
        pltpu.VMEM((GWORDS,), jnp.float32),
        pltpu.VMEM((GWORDS,), jnp.float32),
        pltpu.SemaphoreType.DMA,
        pltpu.SemaphoreType.DMA,
        pltpu.SemaphoreType.DMA,
        pltpu.SemaphoreType.DMA,
    ],
)
def _relayout(wtT_hbm, out_hbm, in0, in1, tr0, tr1, s0, s1, s2, s3):
    _phase_a_body(wtT_hbm, out_hbm, in0, in1, tr0, tr1, s0, s1, s2, s3)


def _phase_b_body(text_hbm, lens_hbm, table_hbm, out_hbm,
                  text_v, lens_v, rows0, rows1, out_v, sem0, sem1):
    wid = lax.axis_index("s") * NC + lax.axis_index("c")
    base = wid * RPW

    pltpu.sync_copy(text_hbm.at[pl.ds(base, RPW), :], text_v)
    pltpu.sync_copy(lens_hbm.at[pl.ds(base, RPW)], lens_v)

    def fire(r, rows_v, sem):
        idx1 = text_v.at[r, pl.ds(0, CH1)]
        idx2 = text_v.at[r, pl.ds(CH1, CH2)]
        pltpu.async_copy(table_hbm.at[idx1], rows_v.at[pl.ds(0, CH1), :], sem)
        pltpu.async_copy(table_hbm.at[idx2], rows_v.at[pl.ds(CH1, CH2), :], sem)

    def wait(r, rows_v, sem):
        idx1 = text_v.at[r, pl.ds(0, CH1)]
        idx2 = text_v.at[r, pl.ds(CH1, CH2)]
        pltpu.make_async_copy(table_hbm.at[idx1],
                              rows_v.at[pl.ds(0, CH1), :], sem).wait()
        pltpu.make_async_copy(table_hbm.at[idx2],
                              rows_v.at[pl.ds(CH1, CH2), :], sem).wait()

    def accumulate(r, rows_v):
        len_vec = plsc.load_gather(lens_v, [jnp.broadcast_to(r, (LANES,))])
        len_s = jnp.max(len_vec)
        n8 = len_s // 8

        def chunk_body(c, carry):
            a0, b0, a1, b1 = carry
            t0 = c * 8
            for u in range(0, 8, 2):
                a0 = a0 + rows_v[t0 + u, 0:16]
                a1 = a1 + rows_v[t0 + u, 16:32]
                b0 = b0 + rows_v[t0 + u + 1, 0:16]
                b1 = b1 + rows_v[t0 + u + 1, 16:32]
            return a0, b0, a1, b1

        zero = jnp.zeros((LANES,), jnp.float32)
        a0, b0, a1, b1 = lax.fori_loop(
            0, n8, chunk_body, (zero, zero, zero, zero))

        def rem_body(t, carry):
            a0, a1 = carry
            return a0 + rows_v[t, 0:16], a1 + rows_v[t, 16:32]

        acc0, acc1 = lax.fori_loop(n8 * 8, len_s, rem_body,
                                   (a0 + b0, a1 + b1))

        inv = 1.0 / len_vec.astype(jnp.float32)
        out_v[r, 0:16] = acc0 * inv
        out_v[r, 16:32] = acc1 * inv

    fire(0, rows0, sem0)

    def outer(i, _):
        r0 = 2 * i
        r1 = 2 * i + 1
        fire(r1, rows1, sem1)
        wait(r0, rows0, sem0)
        accumulate(r0, rows0)

        @pl.when(i < RPW // 2 - 1)
        def _():
            fire(r0 + 2, rows0, sem0)

        wait(r1, rows1, sem1)
        accumulate(r1, rows1)
        return 0

    lax.fori_loop(0, RPW // 2, outer, 0)

    pltpu.sync_copy(out_v, out_hbm.at[pl.ds(base, RPW), :])


@functools.partial(
    pl.kernel,
    out_type=jax.ShapeDtypeStruct((B, D), jnp.float32),
    mesh=plsc.VectorSubcoreMesh(core_axis_name="c", subcore_axis_name="s"),
    compiler_params=pltpu.CompilerParams(
        use_tc_tiling_on_sc=False, needs_layout_passes=False),
    scratch_types=[
        pltpu.VMEM((RPW, SEQ), jnp.int32),
        pltpu.VMEM((RPW,), jnp.int32),
        pltpu.VMEM((SEQ, D), jnp.float32),
        pltpu.VMEM((SEQ, D), jnp.float32),
        pltpu.VMEM((RPW, D), jnp.float32),
        pltpu.SemaphoreType.DMA,
        pltpu.SemaphoreType.DMA,
    ],
)
def _encode(text_hbm, lens_hbm, table_hbm, out_hbm,
            text_v, lens_v, rows0, rows1, out_v, sem0, sem1):
    _phase_b_body(text_hbm, lens_hbm, table_hbm, out_hbm,
                  text_v, lens_v, rows0, rows1, out_v, sem0, sem1)


def kernel(text, text_len, emb_weight):
    wt_lin = _relayout(emb_weight.T).reshape(VOCAB, D)
    return _encode(text.astype(jnp.int32), text_len, wt_lin)
